# baseline (speedup 1.0000x reference)
import functools

import jax
import jax.numpy as jnp
from jax import lax
from jax.experimental import pallas as pl
from jax.experimental.pallas import tpu as pltpu
from jax.experimental.pallas import tpu_sc as plsc

N = 10000
E = 320000
D = 128

NC = 2
NS = 16
NW = NC * NS
C = 112
SK = 5
NCH0 = 115
NCH1 = 65
NSC0 = NCH0 // SK
NSC1 = NCH1 // SK
NCHT = NS * (NCH0 + NCH1)
EPW = 90 * C
E_PAD = NCHT * C
NB = 3

N_PAD = 10240
R = N_PAD // NS
CB = 80
RB = R // CB

_mesh = plsc.VectorSubcoreMesh(core_axis_name="c", subcore_axis_name="s")


def _deg_body(dst_hbm, w_hbm, zn_hbm, out_hbm, didx_v, wv_v, dbuf, acc, sem):
    cid = lax.axis_index("c")
    sid = lax.axis_index("s")
    wid = cid * NS + sid

    pltpu.sync_copy(dst_hbm.at[pl.ds(wid * 90, 90)], didx_v)
    pltpu.sync_copy(w_hbm.at[pl.ds(wid * EPW, EPW)], wv_v)

    pltpu.sync_copy(zn_hbm, dbuf)
    pltpu.sync_copy(dbuf, acc.at[pl.ds(sid * R, R)])
    plsc.subcore_barrier()

    def fire(i, carry):
        pltpu.async_copy(wv_v.at[pl.ds(i * C, C)], acc.at[didx_v.at[i, 0]],
                         sem, add=True)
        return carry

    lax.fori_loop(0, 90, fire, 0)

    def drain(i, carry):
        pltpu.make_async_copy(wv_v.at[pl.ds(i * C, C)], acc.at[didx_v.at[i, 0]],
                              sem).wait()
        return carry

    lax.fori_loop(0, 90, drain, 0)
    plsc.subcore_barrier()

    pltpu.sync_copy(acc.at[pl.ds(sid * R, R)], dbuf)
    pltpu.sync_copy(dbuf, out_hbm.at[pl.ds(cid * N_PAD + sid * R, R)])


_deg_kernel = functools.partial(
    pl.kernel,
    out_type=jax.ShapeDtypeStruct((NC * N_PAD,), jnp.float32),
    mesh=_mesh,
    scratch_types=[
        pltpu.VMEM((90, 1, C), jnp.int32),
        pltpu.VMEM((EPW,), jnp.float32),
        pltpu.VMEM((R,), jnp.float32),
        pltpu.VMEM_SHARED((N_PAD,), jnp.float32),
        pltpu.SemaphoreType.DMA,
    ],
)(_deg_body)


def _msg_body(src_hbm, dst_hbm, w_hbm, h2_hbm, z2_hbm, out_hbm,
              sidx, didx, wv, rows, sem_i, sem_g, sem_s, acc):
    cid = lax.axis_index("c")
    sid = lax.axis_index("s")
    nch = jnp.where(cid == 0, NCH0, NCH1)
    nsc = jnp.where(cid == 0, NSC0, NSC1)
    cb = cid * NS * NCH0 + sid * nch
    eb = cb * C

    def idx_load(k, bb):
        pltpu.async_copy(src_hbm.at[pl.ds(cb + k * SK, SK)],
                         sidx.at[bb], sem_i.at[bb])
        pltpu.async_copy(dst_hbm.at[pl.ds(cb + k * SK, SK)],
                         didx.at[bb], sem_i.at[bb])
        pltpu.async_copy(w_hbm.at[pl.ds(eb + k * SK * C, SK * C)],
                         wv.at[pl.ds(bb * SK * C, SK * C)], sem_i.at[bb])

    def idx_wait(k, bb):
        pltpu.make_async_copy(src_hbm.at[pl.ds(cb + k * SK, SK)],
                              sidx.at[bb], sem_i.at[bb]).wait()
        pltpu.make_async_copy(dst_hbm.at[pl.ds(cb + k * SK, SK)],
                              didx.at[bb], sem_i.at[bb]).wait()
        pltpu.make_async_copy(w_hbm.at[pl.ds(eb + k * SK * C, SK * C)],
                              wv.at[pl.ds(bb * SK * C, SK * C)],
                              sem_i.at[bb]).wait()

    def gather(i, bb, j):
        b = lax.rem(i, NB)
        pltpu.async_copy(h2_hbm.at[sidx.at[bb, j, 0]], rows.at[b],
                         sem_g.at[b])

    def gather_wait(i, bb, j):
        b = lax.rem(i, NB)
        pltpu.make_async_copy(h2_hbm.at[sidx.at[bb, j, 0]], rows.at[b],
                              sem_g.at[b]).wait()

    def scatter(i, bb, j):
        b = lax.rem(i, NB)
        pltpu.async_copy(rows.at[b], acc.at[didx.at[bb, j, 0]], sem_s.at[b],
                         add=True)

    def scatter_wait(i, bb, j):
        b = lax.rem(i, NB)
        pltpu.make_async_copy(rows.at[b], acc.at[didx.at[bb, j, 0]],
                              sem_s.at[b]).wait()

    zslc = rows.at[0].at[pl.ds(0, CB)]
    pltpu.sync_copy(z2_hbm, zslc)
    for t in range(RB):
        pltpu.sync_copy(zslc, acc.at[pl.ds(sid * R + t * CB, CB)])
    plsc.subcore_barrier()

    idx_load(0, 0)
    idx_wait(0, 0)
    idx_load(1, 1)
    gather(0, 0, 0)

    def chunk(i, carry):
        j = lax.rem(i, SK)
        k = lax.div(i, SK)
        bb = lax.rem(k, 2)

        @pl.when(i >= 2)
        def _():
            i2 = i - 2
            scatter_wait(i2, lax.rem(lax.div(i2, SK), 2), lax.rem(i2, SK))

        @pl.when((j == 2) & (k >= 1) & (k + 1 < nsc))
        def _():
            idx_load(k + 1, 1 - bb)

        @pl.when((j < SK - 1) & (i + 1 < nch))
        def _():
            gather(i + 1, bb, j + 1)

        @pl.when((j == SK - 1) & (i + 1 < nch))
        def _():
            idx_wait(k + 1, 1 - bb)
            gather(i + 1, 1 - bb, 0)

        gather_wait(i, bb, j)

        b = lax.rem(i, NB)
        rb = rows.at[b]

        @plsc.parallel_loop(0, C // 16, 1, unroll=2)
        def _(g):
            w16 = wv[pl.ds(bb * SK * C + j * C + g * 16, 16)]
            for jj in range(16):
                w = w16[jj]
                row = rb.at[g * 16 + jj]
                for kk in range(D // 16):
                    sl = pl.ds(kk * 16, 16)
                    row[sl] = row[sl] * w

        scatter(i, bb, j)
        return carry

    lax.fori_loop(0, nch, chunk, 0)

    i2 = nch - 2
    scatter_wait(i2, lax.rem(lax.div(i2, SK), 2), lax.rem(i2, SK))
    i2 = nch - 1
    scatter_wait(i2, lax.rem(lax.div(i2, SK), 2), lax.rem(i2, SK))
    plsc.subcore_barrier()

    for t in range(RB):
        pltpu.sync_copy(acc.at[pl.ds(sid * R + t * CB, CB)], zslc)
        pltpu.sync_copy(zslc,
                        out_hbm.at[pl.ds(cid * N_PAD + sid * R + t * CB, CB)])


_msg_kernel = functools.partial(
    pl.kernel,
    out_type=jax.ShapeDtypeStruct((NC * N_PAD, D), jnp.float32),
    mesh=_mesh,
    scratch_types=[
        pltpu.VMEM((2, SK, 1, C), jnp.int32),
        pltpu.VMEM((2, SK, 1, C), jnp.int32),
        pltpu.VMEM((2 * SK * C,), jnp.float32),
        pltpu.VMEM((NB, C, D), jnp.float32),
        pltpu.SemaphoreType.DMA((2,)),
        pltpu.SemaphoreType.DMA((NB,)),
        pltpu.SemaphoreType.DMA((NB,)),
        pltpu.VMEM_SHARED((N_PAD, D), jnp.float32),
    ],
)(_msg_body)


def _tc1_body(x_ref, w_ref, dp_ref, h2_ref, dinv_ref):
    h = jnp.dot(x_ref[...], w_ref[...], preferred_element_type=jnp.float32)
    deg = dp_ref[pl.ds(0, N), :] + dp_ref[pl.ds(N_PAD, N), :] + 1.0
    dinv = lax.rsqrt(deg)
    dinv_ref[...] = dinv
    h2_ref[...] = h * dinv


def _tc1(x, W, dp):
    return pl.pallas_call(
        _tc1_body,
        out_shape=(
            jax.ShapeDtypeStruct((N, D), jnp.float32),
            jax.ShapeDtypeStruct((N, 1), jnp.float32),
        ),
    )(x, W, dp)


def _tc2_body(s_ref, h2_ref, dinv_ref, x_ref, b_ref, g_ref, be_ref, o_ref):
    s = s_ref[pl.ds(0, N), :] + s_ref[pl.ds(N_PAD, N), :]
    tot = dinv_ref[...] * (s + h2_ref[...]) + b_ref[...]
    mean = jnp.mean(tot, axis=0, keepdims=True)
    ctr = tot - mean
    var = jnp.mean(ctr * ctr, axis=0, keepdims=True)
    bn = ctr * lax.rsqrt(var + 1e-5) * g_ref[...] + be_ref[...]
    o_ref[...] = jnp.maximum(bn + x_ref[...], 0.0)


def _tc2(s, h2, dinv, x, b, gamma, beta):
    return pl.pallas_call(
        _tc2_body,
        out_shape=jax.ShapeDtypeStruct((N, D), jnp.float32),
    )(s, h2, dinv, x, b.reshape(1, D), gamma.reshape(1, D), beta.reshape(1, D))


def kernel(x, edge_index, edge_weight, W, b, gamma, beta):
    src = edge_index[0]
    dst = edge_index[1]
    pad = E_PAD - E
    if pad:
        zpad_i = jnp.zeros((pad,), jnp.int32)
        src = jnp.concatenate([src, zpad_i])
        dst = jnp.concatenate([dst, zpad_i])
        edge_weight = jnp.concatenate([edge_weight, jnp.zeros((pad,), jnp.float32)])

    src3 = src.reshape(NCHT, 1, C)
    dst3 = dst.reshape(NCHT, 1, C)

    zn = jnp.zeros((R,), jnp.float32)
    z2 = jnp.zeros((CB, D), jnp.float32)

    dp = _deg_kernel(dst3, edge_weight, zn)
    h2, dinv = _tc1(x, W, dp.reshape(NC * N_PAD, 1))
    s = _msg_kernel(src3, dst3, edge_weight, h2, z2)
    return _tc2(s, h2, dinv, x, b, gamma, beta)


# speedup vs baseline: 1.0023x; 1.0023x over previous
import functools

import jax
import jax.numpy as jnp
from jax import lax
from jax.experimental import pallas as pl
from jax.experimental.pallas import tpu as pltpu
from jax.experimental.pallas import tpu_sc as plsc

N = 10000
E = 320000
D = 128

NC = 2
NS = 16
NW = NC * NS
C = 112
SK = 5
NCH0 = 110
NCH1 = 70
NSC0 = NCH0 // SK
NSC1 = NCH1 // SK
NCHT = NS * (NCH0 + NCH1)
EPW = 90 * C
E_PAD = NCHT * C
NB = 3

N_PAD = 10240
R = N_PAD // NS
CB = 80
RB = R // CB

_mesh = plsc.VectorSubcoreMesh(core_axis_name="c", subcore_axis_name="s")


def _deg_body(dst_hbm, w_hbm, zn_hbm, out_hbm, didx_v, wv_v, dbuf, acc, sem):
    cid = lax.axis_index("c")
    sid = lax.axis_index("s")
    wid = cid * NS + sid

    pltpu.sync_copy(dst_hbm.at[pl.ds(wid * 90, 90)], didx_v)
    pltpu.sync_copy(w_hbm.at[pl.ds(wid * EPW, EPW)], wv_v)

    pltpu.sync_copy(zn_hbm, dbuf)
    pltpu.sync_copy(dbuf, acc.at[pl.ds(sid * R, R)])
    plsc.subcore_barrier()

    def fire(i, carry):
        pltpu.async_copy(wv_v.at[pl.ds(i * C, C)], acc.at[didx_v.at[i, 0]],
                         sem, add=True)
        return carry

    lax.fori_loop(0, 90, fire, 0)

    def drain(i, carry):
        pltpu.make_async_copy(wv_v.at[pl.ds(i * C, C)], acc.at[didx_v.at[i, 0]],
                              sem).wait()
        return carry

    lax.fori_loop(0, 90, drain, 0)
    plsc.subcore_barrier()

    pltpu.sync_copy(acc.at[pl.ds(sid * R, R)], dbuf)
    pltpu.sync_copy(dbuf, out_hbm.at[pl.ds(cid * N_PAD + sid * R, R)])


_deg_kernel = functools.partial(
    pl.kernel,
    out_type=jax.ShapeDtypeStruct((NC * N_PAD,), jnp.float32),
    mesh=_mesh,
    scratch_types=[
        pltpu.VMEM((90, 1, C), jnp.int32),
        pltpu.VMEM((EPW,), jnp.float32),
        pltpu.VMEM((R,), jnp.float32),
        pltpu.VMEM_SHARED((N_PAD,), jnp.float32),
        pltpu.SemaphoreType.DMA,
    ],
)(_deg_body)


def _msg_body(src_hbm, dst_hbm, w_hbm, h2_hbm, z2_hbm, out_hbm,
              sidx, didx, wv, rows, sem_i, sem_g, sem_s, acc):
    cid = lax.axis_index("c")
    sid = lax.axis_index("s")
    nch = jnp.where(cid == 0, NCH0, NCH1)
    nsc = jnp.where(cid == 0, NSC0, NSC1)
    cb = cid * NS * NCH0 + sid * nch
    eb = cb * C

    def idx_load(k, bb):
        pltpu.async_copy(src_hbm.at[pl.ds(cb + k * SK, SK)],
                         sidx.at[bb], sem_i.at[bb])
        pltpu.async_copy(dst_hbm.at[pl.ds(cb + k * SK, SK)],
                         didx.at[bb], sem_i.at[bb])
        pltpu.async_copy(w_hbm.at[pl.ds(eb + k * SK * C, SK * C)],
                         wv.at[pl.ds(bb * SK * C, SK * C)], sem_i.at[bb])

    def idx_wait(k, bb):
        pltpu.make_async_copy(src_hbm.at[pl.ds(cb + k * SK, SK)],
                              sidx.at[bb], sem_i.at[bb]).wait()
        pltpu.make_async_copy(dst_hbm.at[pl.ds(cb + k * SK, SK)],
                              didx.at[bb], sem_i.at[bb]).wait()
        pltpu.make_async_copy(w_hbm.at[pl.ds(eb + k * SK * C, SK * C)],
                              wv.at[pl.ds(bb * SK * C, SK * C)],
                              sem_i.at[bb]).wait()

    def gather(i, bb, j):
        b = lax.rem(i, NB)
        pltpu.async_copy(h2_hbm.at[sidx.at[bb, j, 0]], rows.at[b],
                         sem_g.at[b])

    def gather_wait(i, bb, j):
        b = lax.rem(i, NB)
        pltpu.make_async_copy(h2_hbm.at[sidx.at[bb, j, 0]], rows.at[b],
                              sem_g.at[b]).wait()

    def scatter(i, bb, j):
        b = lax.rem(i, NB)
        pltpu.async_copy(rows.at[b], acc.at[didx.at[bb, j, 0]], sem_s.at[b],
                         add=True)

    def scatter_wait(i, bb, j):
        b = lax.rem(i, NB)
        pltpu.make_async_copy(rows.at[b], acc.at[didx.at[bb, j, 0]],
                              sem_s.at[b]).wait()

    zslc = rows.at[0].at[pl.ds(0, CB)]
    pltpu.sync_copy(z2_hbm, zslc)
    for t in range(RB):
        pltpu.sync_copy(zslc, acc.at[pl.ds(sid * R + t * CB, CB)])
    plsc.subcore_barrier()

    idx_load(0, 0)
    idx_wait(0, 0)
    idx_load(1, 1)
    gather(0, 0, 0)

    def chunk(i, carry):
        j = lax.rem(i, SK)
        k = lax.div(i, SK)
        bb = lax.rem(k, 2)

        @pl.when(i >= 2)
        def _():
            i2 = i - 2
            scatter_wait(i2, lax.rem(lax.div(i2, SK), 2), lax.rem(i2, SK))

        @pl.when((j == 2) & (k >= 1) & (k + 1 < nsc))
        def _():
            idx_load(k + 1, 1 - bb)

        @pl.when((j < SK - 1) & (i + 1 < nch))
        def _():
            gather(i + 1, bb, j + 1)

        @pl.when((j == SK - 1) & (i + 1 < nch))
        def _():
            idx_wait(k + 1, 1 - bb)
            gather(i + 1, 1 - bb, 0)

        gather_wait(i, bb, j)

        b = lax.rem(i, NB)
        rb = rows.at[b]

        @plsc.parallel_loop(0, C // 16, 1, unroll=2)
        def _(g):
            w16 = wv[pl.ds(bb * SK * C + j * C + g * 16, 16)]
            for jj in range(16):
                w = w16[jj]
                row = rb.at[g * 16 + jj]
                for kk in range(D // 16):
                    sl = pl.ds(kk * 16, 16)
                    row[sl] = row[sl] * w

        scatter(i, bb, j)
        return carry

    lax.fori_loop(0, nch, chunk, 0)

    i2 = nch - 2
    scatter_wait(i2, lax.rem(lax.div(i2, SK), 2), lax.rem(i2, SK))
    i2 = nch - 1
    scatter_wait(i2, lax.rem(lax.div(i2, SK), 2), lax.rem(i2, SK))
    plsc.subcore_barrier()

    for t in range(RB):
        pltpu.sync_copy(acc.at[pl.ds(sid * R + t * CB, CB)], zslc)
        pltpu.sync_copy(zslc,
                        out_hbm.at[pl.ds(cid * N_PAD + sid * R + t * CB, CB)])


_msg_kernel = functools.partial(
    pl.kernel,
    out_type=jax.ShapeDtypeStruct((NC * N_PAD, D), jnp.float32),
    mesh=_mesh,
    scratch_types=[
        pltpu.VMEM((2, SK, 1, C), jnp.int32),
        pltpu.VMEM((2, SK, 1, C), jnp.int32),
        pltpu.VMEM((2 * SK * C,), jnp.float32),
        pltpu.VMEM((NB, C, D), jnp.float32),
        pltpu.SemaphoreType.DMA((2,)),
        pltpu.SemaphoreType.DMA((NB,)),
        pltpu.SemaphoreType.DMA((NB,)),
        pltpu.VMEM_SHARED((N_PAD, D), jnp.float32),
    ],
)(_msg_body)


def _tc1_body(x_ref, w_ref, dp_ref, h2_ref, dinv_ref):
    h = jnp.dot(x_ref[...], w_ref[...], preferred_element_type=jnp.float32)
    deg = dp_ref[pl.ds(0, N), :] + dp_ref[pl.ds(N_PAD, N), :] + 1.0
    dinv = lax.rsqrt(deg)
    dinv_ref[...] = dinv
    h2_ref[...] = h * dinv


def _tc1(x, W, dp):
    return pl.pallas_call(
        _tc1_body,
        out_shape=(
            jax.ShapeDtypeStruct((N, D), jnp.float32),
            jax.ShapeDtypeStruct((N, 1), jnp.float32),
        ),
    )(x, W, dp)


def _tc2_body(s_ref, h2_ref, dinv_ref, x_ref, b_ref, g_ref, be_ref, o_ref):
    s = s_ref[pl.ds(0, N), :] + s_ref[pl.ds(N_PAD, N), :]
    tot = dinv_ref[...] * (s + h2_ref[...]) + b_ref[...]
    mean = jnp.mean(tot, axis=0, keepdims=True)
    ctr = tot - mean
    var = jnp.mean(ctr * ctr, axis=0, keepdims=True)
    bn = ctr * lax.rsqrt(var + 1e-5) * g_ref[...] + be_ref[...]
    o_ref[...] = jnp.maximum(bn + x_ref[...], 0.0)


def _tc2(s, h2, dinv, x, b, gamma, beta):
    return pl.pallas_call(
        _tc2_body,
        out_shape=jax.ShapeDtypeStruct((N, D), jnp.float32),
    )(s, h2, dinv, x, b.reshape(1, D), gamma.reshape(1, D), beta.reshape(1, D))


def kernel(x, edge_index, edge_weight, W, b, gamma, beta):
    src = edge_index[0]
    dst = edge_index[1]
    pad = E_PAD - E
    if pad:
        zpad_i = jnp.zeros((pad,), jnp.int32)
        src = jnp.concatenate([src, zpad_i])
        dst = jnp.concatenate([dst, zpad_i])
        edge_weight = jnp.concatenate([edge_weight, jnp.zeros((pad,), jnp.float32)])

    src3 = src.reshape(NCHT, 1, C)
    dst3 = dst.reshape(NCHT, 1, C)

    zn = jnp.zeros((R,), jnp.float32)
    z2 = jnp.zeros((CB, D), jnp.float32)

    dp = _deg_kernel(dst3, edge_weight, zn)
    h2, dinv = _tc1(x, W, dp.reshape(NC * N_PAD, 1))
    s = _msg_kernel(src3, dst3, edge_weight, h2, z2)
    return _tc2(s, h2, dinv, x, b, gamma, beta)


# speedup vs baseline: 1.0290x; 1.0266x over previous
import functools

import jax
import jax.numpy as jnp
from jax import lax
from jax.experimental import pallas as pl
from jax.experimental.pallas import tpu as pltpu
from jax.experimental.pallas import tpu_sc as plsc

N = 10000
E = 320000
D = 128

NC = 2
NS = 16
NW = NC * NS
C = 112
SK = 5
NCH0 = 110
NCH1 = 70
NSC0 = NCH0 // SK
NSC1 = NCH1 // SK
NCHT = NS * (NCH0 + NCH1)
EPW = 90 * C
E_PAD = NCHT * C
NB = 3

N_PAD = 10240
R = N_PAD // NS
CB = 80
RB = R // CB

_mesh = plsc.VectorSubcoreMesh(core_axis_name="c", subcore_axis_name="s")


def _deg_body(dst_hbm, w_hbm, zn_hbm, out_hbm, didx_v, wv_v, dbuf, acc, sem):
    cid = lax.axis_index("c")
    sid = lax.axis_index("s")
    wid = cid * NS + sid

    pltpu.sync_copy(dst_hbm.at[pl.ds(wid * 90, 90)], didx_v)
    pltpu.sync_copy(w_hbm.at[pl.ds(wid * EPW, EPW)], wv_v)

    pltpu.sync_copy(zn_hbm, dbuf)
    pltpu.sync_copy(dbuf, acc.at[pl.ds(sid * R, R)])
    plsc.subcore_barrier()

    def fire(i, carry):
        pltpu.async_copy(wv_v.at[pl.ds(i * C, C)], acc.at[didx_v.at[i, 0]],
                         sem, add=True)
        return carry

    lax.fori_loop(0, 90, fire, 0)

    def drain(i, carry):
        pltpu.make_async_copy(wv_v.at[pl.ds(i * C, C)], acc.at[didx_v.at[i, 0]],
                              sem).wait()
        return carry

    lax.fori_loop(0, 90, drain, 0)
    plsc.subcore_barrier()

    pltpu.sync_copy(acc.at[pl.ds(sid * R, R)], dbuf)
    pltpu.sync_copy(dbuf, out_hbm.at[pl.ds(cid * N_PAD + sid * R, R)])


_deg_kernel = functools.partial(
    pl.kernel,
    out_type=jax.ShapeDtypeStruct((NC * N_PAD,), jnp.float32),
    mesh=_mesh,
    scratch_types=[
        pltpu.VMEM((90, 1, C), jnp.int32),
        pltpu.VMEM((EPW,), jnp.float32),
        pltpu.VMEM((R,), jnp.float32),
        pltpu.VMEM_SHARED((N_PAD,), jnp.float32),
        pltpu.SemaphoreType.DMA,
    ],
)(_deg_body)


def _msg_body(src_hbm, dst_hbm, w_hbm, h2_hbm, z2_hbm, out_hbm,
              sidx, didx, wv, rows, sem_i, sem_g, sem_s, acc):
    cid = lax.axis_index("c")
    sid = lax.axis_index("s")
    nch = jnp.where(cid == 0, NCH0, NCH1)
    nsc = jnp.where(cid == 0, NSC0, NSC1)
    cb = cid * NS * NCH0 + sid * nch
    eb = cb * C

    def idx_load(k, bb):
        pltpu.async_copy(src_hbm.at[pl.ds(cb + k * SK, SK)],
                         sidx.at[bb], sem_i.at[bb])
        pltpu.async_copy(dst_hbm.at[pl.ds(cb + k * SK, SK)],
                         didx.at[bb], sem_i.at[bb])
        pltpu.async_copy(w_hbm.at[pl.ds(eb + k * SK * C, SK * C)],
                         wv.at[pl.ds(bb * SK * C, SK * C)], sem_i.at[bb])

    def idx_wait(k, bb):
        pltpu.make_async_copy(src_hbm.at[pl.ds(cb + k * SK, SK)],
                              sidx.at[bb], sem_i.at[bb]).wait()
        pltpu.make_async_copy(dst_hbm.at[pl.ds(cb + k * SK, SK)],
                              didx.at[bb], sem_i.at[bb]).wait()
        pltpu.make_async_copy(w_hbm.at[pl.ds(eb + k * SK * C, SK * C)],
                              wv.at[pl.ds(bb * SK * C, SK * C)],
                              sem_i.at[bb]).wait()

    def gather(i, bb, j):
        b = lax.rem(i, NB)
        pltpu.async_copy(h2_hbm.at[sidx.at[bb, j, 0]], rows.at[b],
                         sem_g.at[b])

    def gather_wait(i, bb, j):
        b = lax.rem(i, NB)
        pltpu.make_async_copy(h2_hbm.at[sidx.at[bb, j, 0]], rows.at[b],
                              sem_g.at[b]).wait()

    def scatter(i, bb, j):
        b = lax.rem(i, NB)
        pltpu.async_copy(rows.at[b], acc.at[didx.at[bb, j, 0]], sem_s.at[b],
                         add=True)

    def scatter_wait(i, bb, j):
        b = lax.rem(i, NB)
        pltpu.make_async_copy(rows.at[b], acc.at[didx.at[bb, j, 0]],
                              sem_s.at[b]).wait()

    zslc = rows.at[0].at[pl.ds(0, CB)]
    pltpu.sync_copy(z2_hbm, zslc)
    for t in range(RB):
        pltpu.sync_copy(zslc, acc.at[pl.ds(sid * R + t * CB, CB)])
    plsc.subcore_barrier()

    idx_load(0, 0)
    idx_wait(0, 0)
    idx_load(1, 1)
    gather(0, 0, 0)

    def chunk(i, carry):
        j = lax.rem(i, SK)
        k = lax.div(i, SK)
        bb = lax.rem(k, 2)

        @pl.when(i >= 2)
        def _():
            i2 = i - 2
            scatter_wait(i2, lax.rem(lax.div(i2, SK), 2), lax.rem(i2, SK))

        @pl.when((j == 2) & (k >= 1) & (k + 1 < nsc))
        def _():
            idx_load(k + 1, 1 - bb)

        @pl.when((j < SK - 1) & (i + 1 < nch))
        def _():
            gather(i + 1, bb, j + 1)

        @pl.when((j == SK - 1) & (i + 1 < nch))
        def _():
            idx_wait(k + 1, 1 - bb)
            gather(i + 1, 1 - bb, 0)

        gather_wait(i, bb, j)

        b = lax.rem(i, NB)
        rb = rows.at[b]

        @plsc.parallel_loop(0, C // 16, 1, unroll=2)
        def _(g):
            w16 = wv[pl.ds(bb * SK * C + j * C + g * 16, 16)]
            for jj in range(16):
                w = w16[jj]
                row = rb.at[g * 16 + jj]
                for kk in range(D // 16):
                    sl = pl.ds(kk * 16, 16)
                    row[sl] = row[sl] * w

        scatter(i, bb, j)
        return carry

    lax.fori_loop(0, nch, chunk, 0)

    i2 = nch - 2
    scatter_wait(i2, lax.rem(lax.div(i2, SK), 2), lax.rem(i2, SK))
    i2 = nch - 1
    scatter_wait(i2, lax.rem(lax.div(i2, SK), 2), lax.rem(i2, SK))
    plsc.subcore_barrier()

    for t in range(RB):
        pltpu.sync_copy(acc.at[pl.ds(sid * R + t * CB, CB)], zslc)
        pltpu.sync_copy(zslc,
                        out_hbm.at[pl.ds(cid * N_PAD + sid * R + t * CB, CB)])


_msg_kernel = functools.partial(
    pl.kernel,
    out_type=jax.ShapeDtypeStruct((NC * N_PAD, D), jnp.float32),
    mesh=_mesh,
    scratch_types=[
        pltpu.VMEM((2, SK, 1, C), jnp.int32),
        pltpu.VMEM((2, SK, 1, C), jnp.int32),
        pltpu.VMEM((2 * SK * C,), jnp.float32),
        pltpu.VMEM((NB, C, D), jnp.float32),
        pltpu.SemaphoreType.DMA((2,)),
        pltpu.SemaphoreType.DMA((NB,)),
        pltpu.SemaphoreType.DMA((NB,)),
        pltpu.VMEM_SHARED((N_PAD, D), jnp.float32),
    ],
)(_msg_body)


def _tc1a_body(x_ref, w_ref, h_ref):
    h_ref[...] = jnp.dot(x_ref[...], w_ref[...],
                         preferred_element_type=jnp.float32)


def _tc1a(x, W):
    return pl.pallas_call(
        _tc1a_body,
        out_shape=jax.ShapeDtypeStruct((N, D), jnp.float32),
    )(x, W)


def _tc1b_body(h_ref, dp_ref, h2_ref, dinv_ref):
    deg = dp_ref[pl.ds(0, N), :] + dp_ref[pl.ds(N_PAD, N), :] + 1.0
    dinv = lax.rsqrt(deg)
    dinv_ref[...] = dinv
    h2_ref[...] = h_ref[...] * dinv


def _tc1b(h, dp):
    return pl.pallas_call(
        _tc1b_body,
        out_shape=(
            jax.ShapeDtypeStruct((N, D), jnp.float32),
            jax.ShapeDtypeStruct((N, 1), jnp.float32),
        ),
    )(h, dp)


def _tc2_body(s_ref, h2_ref, dinv_ref, x_ref, b_ref, g_ref, be_ref, o_ref):
    s = s_ref[pl.ds(0, N), :] + s_ref[pl.ds(N_PAD, N), :]
    tot = dinv_ref[...] * (s + h2_ref[...]) + b_ref[...]
    mean = jnp.mean(tot, axis=0, keepdims=True)
    ctr = tot - mean
    var = jnp.mean(ctr * ctr, axis=0, keepdims=True)
    bn = ctr * lax.rsqrt(var + 1e-5) * g_ref[...] + be_ref[...]
    o_ref[...] = jnp.maximum(bn + x_ref[...], 0.0)


def _tc2(s, h2, dinv, x, b, gamma, beta):
    return pl.pallas_call(
        _tc2_body,
        out_shape=jax.ShapeDtypeStruct((N, D), jnp.float32),
    )(s, h2, dinv, x, b.reshape(1, D), gamma.reshape(1, D), beta.reshape(1, D))


def kernel(x, edge_index, edge_weight, W, b, gamma, beta):
    src = edge_index[0]
    dst = edge_index[1]
    pad = E_PAD - E
    if pad:
        zpad_i = jnp.zeros((pad,), jnp.int32)
        src = jnp.concatenate([src, zpad_i])
        dst = jnp.concatenate([dst, zpad_i])
        edge_weight = jnp.concatenate([edge_weight, jnp.zeros((pad,), jnp.float32)])

    src3 = src.reshape(NCHT, 1, C)
    dst3 = dst.reshape(NCHT, 1, C)

    zn = jnp.zeros((R,), jnp.float32)
    z2 = jnp.zeros((CB, D), jnp.float32)

    h = _tc1a(x, W)
    dp = _deg_kernel(dst3, edge_weight, zn)
    h2, dinv = _tc1b(h, dp.reshape(NC * N_PAD, 1))
    s = _msg_kernel(src3, dst3, edge_weight, h2, z2)
    return _tc2(s, h2, dinv, x, b, gamma, beta)
